# baseline (speedup 1.0000x reference)
import jax
import jax.numpy as jnp
from jax import lax
from jax.experimental import pallas as pl
from jax.experimental.pallas import tpu as pltpu
from jax.experimental.pallas import tpu_sc as plsc

N = 10000
E = 320000
D = 128

NC = 2
NS = 16
NW = NC * NS

CHUNK = 128
NCH0 = 90
NCH1 = 67
MAXNCH = max(NCH0, NCH1)
E0 = NS * NCH0 * CHUNK
E1 = NS * NCH1 * CHUNK
E_PAD = E0 + E1
N_PAD = 10112
ROWS_PER_TILE = N_PAD // NS


def _sc_body(x_hbm, row0_hbm, col0_hbm, row1_hbm, col1_hbm, zero_hbm, out_hbm,
             row_v, col_v, buf, sem, agg_sh):
    cid = lax.axis_index("c")
    sid = lax.axis_index("s")

    pltpu.sync_copy(
        zero_hbm.at[pl.ds(sid * ROWS_PER_TILE, ROWS_PER_TILE)],
        agg_sh.at[pl.ds(sid * ROWS_PER_TILE, ROWS_PER_TILE)],
    )
    plsc.subcore_barrier()

    def run(row_hbm, col_hbm, nch):
        pltpu.sync_copy(row_hbm.at[sid], row_v.at[pl.ds(0, nch)])
        pltpu.sync_copy(col_hbm.at[sid], col_v.at[pl.ds(0, nch)])

        def step(i, c):
            pltpu.async_copy(x_hbm.at[col_v.at[i]], buf, sem).wait()
            pltpu.sync_copy(buf, agg_sh.at[row_v.at[i]], add=True)
            return c

        lax.fori_loop(0, nch, step, 0)

    @pl.when(cid == 0)
    def _():
        run(row0_hbm, col0_hbm, NCH0)

    @pl.when(cid == 1)
    def _():
        run(row1_hbm, col1_hbm, NCH1)

    plsc.subcore_barrier()

    pltpu.sync_copy(
        agg_sh.at[pl.ds(sid * ROWS_PER_TILE, ROWS_PER_TILE)],
        out_hbm.at[cid, pl.ds(sid * ROWS_PER_TILE, ROWS_PER_TILE)],
    )


@jax.jit
def _sc_aggregate(x, row0, col0, row1, col1, zeros):
    mesh = plsc.VectorSubcoreMesh(core_axis_name="c", subcore_axis_name="s")
    return pl.kernel(
        _sc_body,
        out_type=jax.ShapeDtypeStruct((NC, N_PAD, D), jnp.float32),
        mesh=mesh,
        scratch_types=[
            pltpu.VMEM((MAXNCH, CHUNK), jnp.int32),
            pltpu.VMEM((MAXNCH, CHUNK), jnp.int32),
            pltpu.VMEM((CHUNK, D), jnp.float32),
            pltpu.SemaphoreType.DMA,
            pltpu.VMEM_SHARED((N_PAD, D), jnp.float32),
        ],
    )(x, row0, col0, row1, col1, zeros)


def _tc_body(x_ref, p_ref, w1_ref, b1_ref, w2_ref, b2_ref, g_ref, bt_ref, o_ref):
    h = x_ref[...] + p_ref[0, :N, :] + p_ref[1, :N, :]
    h = lax.dot_general(h, w1_ref[...], (((1,), (1,)), ((), ())),
                        preferred_element_type=jnp.float32) + b1_ref[...]
    h = jnp.maximum(h, 0.0)
    h = lax.dot_general(h, w2_ref[...], (((1,), (1,)), ((), ())),
                        preferred_element_type=jnp.float32) + b2_ref[...]
    mean = jnp.mean(h, axis=0)
    var = jnp.mean(h * h, axis=0) - mean * mean
    o_ref[...] = (h - mean) * lax.rsqrt(var + 1e-5) * g_ref[...] + bt_ref[...]


@jax.jit
def _tc_mlp_bn(x, partials, W1, b1, W2, b2, gamma, beta):
    return pl.pallas_call(
        _tc_body,
        out_shape=jax.ShapeDtypeStruct((N, D), jnp.float32),
    )(x, partials, W1, b1, W2, b2, gamma, beta)


def kernel(x, edge_index, W1, b1, W2, b2, gamma, beta):
    row = edge_index[0].astype(jnp.int32)
    col = edge_index[1].astype(jnp.int32)
    pad = E_PAD - E
    row_p = jnp.concatenate([row, jnp.full((pad,), N, jnp.int32)])
    col_p = jnp.concatenate([col, jnp.zeros((pad,), jnp.int32)])
    row0 = row_p[:E0].reshape(NS, NCH0, CHUNK)
    col0 = col_p[:E0].reshape(NS, NCH0, CHUNK)
    row1 = row_p[E0:].reshape(NS, NCH1, CHUNK)
    col1 = col_p[E0:].reshape(NS, NCH1, CHUNK)
    zeros = jnp.zeros((N_PAD, D), jnp.float32)
    partials = _sc_aggregate(x, row0, col0, row1, col1, zeros)
    return _tc_mlp_bn(x, partials, W1, b1, W2, b2, gamma, beta)


# speedup vs baseline: 1.0177x; 1.0177x over previous
import jax
import jax.numpy as jnp
from jax import lax
from jax.experimental import pallas as pl
from jax.experimental.pallas import tpu as pltpu
from jax.experimental.pallas import tpu_sc as plsc

N = 10000
E = 320000
D = 128

NC = 2
NS = 16
NW = NC * NS

CHUNK = 128
NCH0 = 92
NCH1 = 65
MAXNCH = max(NCH0, NCH1)
E0 = NS * NCH0 * CHUNK
E1 = NS * NCH1 * CHUNK
E_PAD = E0 + E1
N_PAD = 10112
ROWS_PER_TILE = N_PAD // NS


def _sc_body(x_hbm, row0_hbm, col0_hbm, row1_hbm, col1_hbm, zero_hbm, out_hbm,
             row_v, col_v, buf, sem, agg_sh):
    cid = lax.axis_index("c")
    sid = lax.axis_index("s")

    pltpu.sync_copy(
        zero_hbm.at[pl.ds(sid * ROWS_PER_TILE, ROWS_PER_TILE)],
        agg_sh.at[pl.ds(sid * ROWS_PER_TILE, ROWS_PER_TILE)],
    )
    plsc.subcore_barrier()

    def run(row_hbm, col_hbm, nch):
        pltpu.sync_copy(row_hbm.at[sid], row_v.at[pl.ds(0, nch)])
        pltpu.sync_copy(col_hbm.at[sid], col_v.at[pl.ds(0, nch)])

        def step(i, c):
            pltpu.async_copy(x_hbm.at[col_v.at[i]], buf, sem).wait()
            pltpu.sync_copy(buf, agg_sh.at[row_v.at[i]], add=True)
            return c

        lax.fori_loop(0, nch, step, 0)

    @pl.when(cid == 0)
    def _():
        run(row0_hbm, col0_hbm, NCH0)

    @pl.when(cid == 1)
    def _():
        run(row1_hbm, col1_hbm, NCH1)

    plsc.subcore_barrier()

    pltpu.sync_copy(
        agg_sh.at[pl.ds(sid * ROWS_PER_TILE, ROWS_PER_TILE)],
        out_hbm.at[cid, pl.ds(sid * ROWS_PER_TILE, ROWS_PER_TILE)],
    )


@jax.jit
def _sc_aggregate(x, row0, col0, row1, col1, zeros):
    mesh = plsc.VectorSubcoreMesh(core_axis_name="c", subcore_axis_name="s")
    return pl.kernel(
        _sc_body,
        out_type=jax.ShapeDtypeStruct((NC, N_PAD, D), jnp.float32),
        mesh=mesh,
        scratch_types=[
            pltpu.VMEM((MAXNCH, CHUNK), jnp.int32),
            pltpu.VMEM((MAXNCH, CHUNK), jnp.int32),
            pltpu.VMEM((CHUNK, D), jnp.float32),
            pltpu.SemaphoreType.DMA,
            pltpu.VMEM_SHARED((N_PAD, D), jnp.float32),
        ],
    )(x, row0, col0, row1, col1, zeros)


def _tc_body(x_ref, p_ref, w1_ref, b1_ref, w2_ref, b2_ref, g_ref, bt_ref, o_ref):
    h = x_ref[...] + p_ref[0, :N, :] + p_ref[1, :N, :]
    h = lax.dot_general(h, w1_ref[...], (((1,), (1,)), ((), ())),
                        preferred_element_type=jnp.float32) + b1_ref[...]
    h = jnp.maximum(h, 0.0)
    h = lax.dot_general(h, w2_ref[...], (((1,), (1,)), ((), ())),
                        preferred_element_type=jnp.float32) + b2_ref[...]
    mean = jnp.mean(h, axis=0)
    var = jnp.mean(h * h, axis=0) - mean * mean
    o_ref[...] = (h - mean) * lax.rsqrt(var + 1e-5) * g_ref[...] + bt_ref[...]


@jax.jit
def _tc_mlp_bn(x, partials, W1, b1, W2, b2, gamma, beta):
    return pl.pallas_call(
        _tc_body,
        out_shape=jax.ShapeDtypeStruct((N, D), jnp.float32),
    )(x, partials, W1, b1, W2, b2, gamma, beta)


def kernel(x, edge_index, W1, b1, W2, b2, gamma, beta):
    row = edge_index[0].astype(jnp.int32)
    col = edge_index[1].astype(jnp.int32)
    pad = E_PAD - E
    row_p = jnp.concatenate([row, jnp.full((pad,), N, jnp.int32)])
    col_p = jnp.concatenate([col, jnp.zeros((pad,), jnp.int32)])
    row0 = row_p[:E0].reshape(NS, NCH0, CHUNK)
    col0 = col_p[:E0].reshape(NS, NCH0, CHUNK)
    row1 = row_p[E0:].reshape(NS, NCH1, CHUNK)
    col1 = col_p[E0:].reshape(NS, NCH1, CHUNK)
    zeros = jnp.zeros((N_PAD, D), jnp.float32)
    partials = _sc_aggregate(x, row0, col0, row1, col1, zeros)
    return _tc_mlp_bn(x, partials, W1, b1, W2, b2, gamma, beta)
